# baseline (speedup 1.0000x reference)
import jax
import jax.numpy as jnp
from jax.experimental import pallas as pl
from jax.experimental.pallas import tpu as pltpu

MAX_COLL = 8
SIGMA = 1e-4
WEIGHT = 0.1

BM = 64
BP = 2048


def _feat_kernel(tri_ref, feat_ref):
    t = tri_ref[...]
    v0 = t[:, 0:3]
    v1 = t[:, 3:6]
    v2 = t[:, 6:9]
    cent = (v0 + v1 + v2) * (1.0 / 3.0)
    bbmin = jnp.minimum(jnp.minimum(v0, v1), v2)
    bbmax = jnp.maximum(jnp.maximum(v0, v1), v2)

    def _n(v):
        d = v - cent
        return jnp.sqrt(jnp.sum(d * d, axis=1, keepdims=True))

    r = jnp.maximum(jnp.maximum(_n(v0), _n(v1)), _n(v2))
    out = jnp.concatenate([cent, r, bbmin, bbmax, jnp.zeros_like(t[:, 0:6])],
                          axis=1)
    feat_ref[...] = out


def _knn_kernel(featr_ref, centt_ref, nbr_ref):
    i = pl.program_id(0)
    fr = featr_ref[...]
    ct = centt_ref[...]
    tp = ct.shape[1]

    dx = fr[:, 0:1] - ct[0:1, :]
    dy = fr[:, 1:2] - ct[1:2, :]
    dz = fr[:, 2:3] - ct[2:3, :]
    d2 = dx * dx + dy * dy + dz * dz

    col_i = jax.lax.broadcasted_iota(jnp.int32, (BM, tp), 1)
    row_i = jax.lax.broadcasted_iota(jnp.int32, (BM, tp), 0) + i * BM
    colf = col_i.astype(jnp.float32)
    inf = jnp.float32(jnp.inf)
    d2 = jnp.where(col_i == row_i, inf, d2)

    big = jnp.float32(1e9)
    for k in range(MAX_COLL):
        mval = jnp.min(d2, axis=1, keepdims=True)
        eq = d2 == mval
        idxf = jnp.min(jnp.where(eq, colf, big), axis=1, keepdims=True)
        nbr_ref[:, k:k + 1] = idxf.astype(jnp.int32)
        d2 = jnp.where(eq, inf, d2)


def _pen_kernel(fp_ref, fq_ref, tp_ref, tq_ref, out_ref):
    i = pl.program_id(0)

    @pl.when(i == 0)
    def _():
        out_ref[...] = jnp.zeros_like(out_ref)

    fp = fp_ref[...]
    fq = fq_ref[...]
    tv = tp_ref[...]
    qv = tq_ref[...]

    rp = fp[3:4, :]
    rq = fq[3:4, :]
    ov = jnp.all((fp[4:7, :] <= fq[7:10, :]) & (fq[4:7, :] <= fp[7:10, :]),
                 axis=0, keepdims=True)
    valid = fp[10:11, :]

    pen = jnp.zeros_like(rp)
    for j in range(3):
        dqx = qv[3 * j + 0:3 * j + 1, :] - fp[0:1, :]
        dqy = qv[3 * j + 1:3 * j + 2, :] - fp[1:2, :]
        dqz = qv[3 * j + 2:3 * j + 3, :] - fp[2:3, :]
        dq = jnp.sqrt(dqx * dqx + dqy * dqy + dqz * dqz)
        pen += jnp.maximum(rp - dq, 0.0)
        dpx = tv[3 * j + 0:3 * j + 1, :] - fq[0:1, :]
        dpy = tv[3 * j + 1:3 * j + 2, :] - fq[1:2, :]
        dpz = tv[3 * j + 2:3 * j + 3, :] - fq[2:3, :]
        dp = jnp.sqrt(dpx * dpx + dpy * dpy + dpz * dpz)
        pen += jnp.maximum(rq - dp, 0.0)

    pen = pen * jnp.where(ov, 1.0, 0.0) * valid * jnp.float32(1.0 / SIGMA)
    out_ref[...] += jnp.sum(pen)


def kernel(verts, trans, faces):
    nv = verts.shape[1]
    nf = faces.shape[0]
    t = 2 * nf
    tp_pad = ((t + BM - 1) // BM) * BM

    vertices = verts + trans[:, None, :]
    flat = vertices.reshape(-1, 3)
    fidx = jnp.concatenate([faces, faces + nv], axis=0)
    tri9 = flat[fidx].reshape(t, 9)
    tri16 = jnp.pad(tri9, ((0, tp_pad - t), (0, 7)),
                    constant_values=1e6)
    tri16 = tri16.at[:t, 9:].set(0.0)

    feat = pl.pallas_call(
        _feat_kernel,
        grid=(tp_pad // BM,),
        in_specs=[pl.BlockSpec((BM, 16), lambda i: (i, 0))],
        out_specs=pl.BlockSpec((BM, 16), lambda i: (i, 0)),
        out_shape=jax.ShapeDtypeStruct((tp_pad, 16), jnp.float32),
        compiler_params=pltpu.CompilerParams(
            dimension_semantics=("parallel",)),
    )(tri16)

    centt = jnp.zeros((8, tp_pad), jnp.float32).at[0:3, :].set(feat[:, 0:3].T)

    nbr = pl.pallas_call(
        _knn_kernel,
        grid=(tp_pad // BM,),
        in_specs=[
            pl.BlockSpec((BM, 16), lambda i: (i, 0)),
            pl.BlockSpec((8, tp_pad), lambda i: (0, 0)),
        ],
        out_specs=pl.BlockSpec((BM, MAX_COLL), lambda i: (i, 0)),
        out_shape=jax.ShapeDtypeStruct((tp_pad, MAX_COLL), jnp.int32),
        compiler_params=pltpu.CompilerParams(
            dimension_semantics=("parallel",)),
    )(feat, centt)

    nbr = nbr[:t]
    q = nbr.reshape(-1)
    np_pairs = t * MAX_COLL
    npp = ((np_pairs + BP - 1) // BP) * BP

    featp = jnp.repeat(feat[:t, 0:16], MAX_COLL, axis=0)
    valid_col = jnp.ones((np_pairs, 1), jnp.float32)
    featp = featp.at[:, 10:11].set(valid_col)
    featq = feat[q]
    trip = jnp.repeat(tri9[:t], MAX_COLL, axis=0)
    triq = tri9[q]

    def _lane_major(a, rows):
        a = jnp.pad(a, ((0, npp - np_pairs), (0, 16 - a.shape[1])))
        return a.T if rows == 16 else a.T

    fpm = _lane_major(featp, 16)
    fqm = _lane_major(featq, 16)
    tpm = _lane_major(trip, 16)
    tqm = _lane_major(triq, 16)

    acc = pl.pallas_call(
        _pen_kernel,
        grid=(npp // BP,),
        in_specs=[
            pl.BlockSpec((16, BP), lambda i: (0, i)),
            pl.BlockSpec((16, BP), lambda i: (0, i)),
            pl.BlockSpec((16, BP), lambda i: (0, i)),
            pl.BlockSpec((16, BP), lambda i: (0, i)),
        ],
        out_specs=pl.BlockSpec((8, 128), lambda i: (0, 0)),
        out_shape=jax.ShapeDtypeStruct((8, 128), jnp.float32),
    )(fpm, fqm, tpm, tqm)

    pen_loss = acc[0, 0]
    mask = (pen_loss < 2000.0).astype(jnp.float32)
    vals = jax.nn.sigmoid(pen_loss / 2000.0) - 0.5
    denom = jnp.maximum(mask, 1.0)
    return vals * mask / denom * WEIGHT


# speedup vs baseline: 1.0180x; 1.0180x over previous
import jax
import jax.numpy as jnp
from jax.experimental import pallas as pl
from jax.experimental.pallas import tpu as pltpu

MAX_COLL = 8
SIGMA = 1e-4
WEIGHT = 0.1

BM = 128
BP = 2048


def _feat_kernel(tri_ref, feat_ref):
    t = tri_ref[...]
    v0 = t[:, 0:3]
    v1 = t[:, 3:6]
    v2 = t[:, 6:9]
    cent = (v0 + v1 + v2) * (1.0 / 3.0)
    bbmin = jnp.minimum(jnp.minimum(v0, v1), v2)
    bbmax = jnp.maximum(jnp.maximum(v0, v1), v2)

    def _n(v):
        d = v - cent
        return jnp.sqrt(jnp.sum(d * d, axis=1, keepdims=True))

    r = jnp.maximum(jnp.maximum(_n(v0), _n(v1)), _n(v2))
    out = jnp.concatenate([cent, r, bbmin, bbmax, jnp.zeros_like(t[:, 0:6])],
                          axis=1)
    feat_ref[...] = out


def _knn_kernel(featr_ref, centt_ref, nbr_ref):
    i = pl.program_id(0)
    fr = featr_ref[...]
    ct = centt_ref[...]
    tp = ct.shape[1]

    dx = fr[:, 0:1] - ct[0:1, :]
    dy = fr[:, 1:2] - ct[1:2, :]
    dz = fr[:, 2:3] - ct[2:3, :]
    d2 = dx * dx + dy * dy + dz * dz

    col_i = jax.lax.broadcasted_iota(jnp.int32, (BM, tp), 1)
    row_i = jax.lax.broadcasted_iota(jnp.int32, (BM, tp), 0) + i * BM
    colf = col_i.astype(jnp.float32)
    inf = jnp.float32(jnp.inf)
    d2 = jnp.where(col_i == row_i, inf, d2)

    big = jnp.float32(1e9)
    for k in range(MAX_COLL):
        mval = jnp.min(d2, axis=1, keepdims=True)
        eq = d2 == mval
        idxf = jnp.min(jnp.where(eq, colf, big), axis=1, keepdims=True)
        nbr_ref[:, k:k + 1] = idxf.astype(jnp.int32)
        d2 = jnp.where(colf == idxf, inf, d2)


def _pen_kernel(fp_ref, fq_ref, tp_ref, tq_ref, out_ref):
    i = pl.program_id(0)

    @pl.when(i == 0)
    def _():
        out_ref[...] = jnp.zeros_like(out_ref)

    fp = fp_ref[...]
    fq = fq_ref[...]
    tv = tp_ref[...]
    qv = tq_ref[...]

    rp = fp[3:4, :]
    rq = fq[3:4, :]
    ov = jnp.all((fp[4:7, :] <= fq[7:10, :]) & (fq[4:7, :] <= fp[7:10, :]),
                 axis=0, keepdims=True)
    valid = fp[10:11, :]

    pen = jnp.zeros_like(rp)
    for j in range(3):
        dqx = qv[3 * j + 0:3 * j + 1, :] - fp[0:1, :]
        dqy = qv[3 * j + 1:3 * j + 2, :] - fp[1:2, :]
        dqz = qv[3 * j + 2:3 * j + 3, :] - fp[2:3, :]
        dq = jnp.sqrt(dqx * dqx + dqy * dqy + dqz * dqz)
        pen += jnp.maximum(rp - dq, 0.0)
        dpx = tv[3 * j + 0:3 * j + 1, :] - fq[0:1, :]
        dpy = tv[3 * j + 1:3 * j + 2, :] - fq[1:2, :]
        dpz = tv[3 * j + 2:3 * j + 3, :] - fq[2:3, :]
        dp = jnp.sqrt(dpx * dpx + dpy * dpy + dpz * dpz)
        pen += jnp.maximum(rq - dp, 0.0)

    pen = pen * jnp.where(ov, 1.0, 0.0) * valid * jnp.float32(1.0 / SIGMA)
    out_ref[...] += jnp.sum(pen)


def kernel(verts, trans, faces):
    nv = verts.shape[1]
    nf = faces.shape[0]
    t = 2 * nf
    tp_pad = ((t + BM - 1) // BM) * BM

    vertices = verts + trans[:, None, :]
    flat = vertices.reshape(-1, 3)
    fidx = jnp.concatenate([faces, faces + nv], axis=0)
    tri9 = flat[fidx].reshape(t, 9)
    tri16 = jnp.pad(tri9, ((0, tp_pad - t), (0, 7)),
                    constant_values=1e6)
    tri16 = tri16.at[:t, 9:].set(0.0)

    feat = pl.pallas_call(
        _feat_kernel,
        grid=(tp_pad // BM,),
        in_specs=[pl.BlockSpec((BM, 16), lambda i: (i, 0))],
        out_specs=pl.BlockSpec((BM, 16), lambda i: (i, 0)),
        out_shape=jax.ShapeDtypeStruct((tp_pad, 16), jnp.float32),
        compiler_params=pltpu.CompilerParams(
            dimension_semantics=("parallel",)),
    )(tri16)

    centt = jnp.zeros((8, tp_pad), jnp.float32).at[0:3, :].set(feat[:, 0:3].T)

    nbr = pl.pallas_call(
        _knn_kernel,
        grid=(tp_pad // BM,),
        in_specs=[
            pl.BlockSpec((BM, 16), lambda i: (i, 0)),
            pl.BlockSpec((8, tp_pad), lambda i: (0, 0)),
        ],
        out_specs=pl.BlockSpec((BM, MAX_COLL), lambda i: (i, 0)),
        out_shape=jax.ShapeDtypeStruct((tp_pad, MAX_COLL), jnp.int32),
        compiler_params=pltpu.CompilerParams(
            dimension_semantics=("parallel",)),
    )(feat, centt)

    nbr = nbr[:t]
    q = nbr.reshape(-1)
    np_pairs = t * MAX_COLL
    npp = ((np_pairs + BP - 1) // BP) * BP

    featp = jnp.repeat(feat[:t, 0:16], MAX_COLL, axis=0)
    valid_col = jnp.ones((np_pairs, 1), jnp.float32)
    featp = featp.at[:, 10:11].set(valid_col)
    featq = feat[q]
    trip = jnp.repeat(tri9[:t], MAX_COLL, axis=0)
    triq = tri9[q]

    def _lane_major(a, rows):
        a = jnp.pad(a, ((0, npp - np_pairs), (0, 16 - a.shape[1])))
        return a.T if rows == 16 else a.T

    fpm = _lane_major(featp, 16)
    fqm = _lane_major(featq, 16)
    tpm = _lane_major(trip, 16)
    tqm = _lane_major(triq, 16)

    acc = pl.pallas_call(
        _pen_kernel,
        grid=(npp // BP,),
        in_specs=[
            pl.BlockSpec((16, BP), lambda i: (0, i)),
            pl.BlockSpec((16, BP), lambda i: (0, i)),
            pl.BlockSpec((16, BP), lambda i: (0, i)),
            pl.BlockSpec((16, BP), lambda i: (0, i)),
        ],
        out_specs=pl.BlockSpec((8, 128), lambda i: (0, 0)),
        out_shape=jax.ShapeDtypeStruct((8, 128), jnp.float32),
    )(fpm, fqm, tpm, tqm)

    pen_loss = acc[0, 0]
    mask = (pen_loss < 2000.0).astype(jnp.float32)
    vals = jax.nn.sigmoid(pen_loss / 2000.0) - 0.5
    denom = jnp.maximum(mask, 1.0)
    return vals * mask / denom * WEIGHT
